# Initial kernel scaffold; baseline (speedup 1.0000x reference)
import jax
import jax.numpy as jnp
from jax.experimental import pallas as pl


def kernel(state_action_values, action, q_prime):
    raise NotImplementedError("write your pallas kernel here")


# speedup vs baseline: 1.3223x; 1.3223x over previous
import jax
import jax.numpy as jnp
from jax.experimental import pallas as pl


def _bellman_block(sav_ref, act_ref, q_ref, out_ref):
    cols = jax.lax.broadcasted_iota(jnp.int32, out_ref.shape, 1)
    out_ref[...] = jnp.where(cols == act_ref[...], q_ref[...], sav_ref[...])


def kernel(state_action_values, action, q_prime):
    B, C = state_action_values.shape
    blk = 512
    q2 = q_prime.reshape(B, 1)
    act = action.astype(jnp.int32)
    return pl.pallas_call(
        _bellman_block,
        grid=(B // blk,),
        in_specs=[
            pl.BlockSpec((blk, C), lambda i: (i, 0)),
            pl.BlockSpec((blk, 1), lambda i: (i, 0)),
            pl.BlockSpec((blk, 1), lambda i: (i, 0)),
        ],
        out_specs=pl.BlockSpec((blk, C), lambda i: (i, 0)),
        out_shape=jax.ShapeDtypeStruct((B, C), state_action_values.dtype),
    )(state_action_values, act, q2)


# speedup vs baseline: 1.3538x; 1.0238x over previous
import jax
import jax.numpy as jnp
from jax.experimental import pallas as pl
from jax.experimental.pallas import tpu as pltpu


def _bellman_block(sav_ref, act_ref, q_ref, out_ref):
    cols = jax.lax.broadcasted_iota(jnp.int32, out_ref.shape, 1)
    out_ref[...] = jnp.where(cols == act_ref[...], q_ref[...], sav_ref[...])


def kernel(state_action_values, action, q_prime):
    B, C = state_action_values.shape
    blk = 2048
    q2 = q_prime.reshape(B, 1)
    act = action.astype(jnp.int32)
    return pl.pallas_call(
        _bellman_block,
        grid=(B // blk,),
        in_specs=[
            pl.BlockSpec((blk, C), lambda i: (i, 0)),
            pl.BlockSpec((blk, 1), lambda i: (i, 0)),
            pl.BlockSpec((blk, 1), lambda i: (i, 0)),
        ],
        out_specs=pl.BlockSpec((blk, C), lambda i: (i, 0)),
        out_shape=jax.ShapeDtypeStruct((B, C), state_action_values.dtype),
        compiler_params=pltpu.CompilerParams(
            dimension_semantics=("parallel",),
        ),
    )(state_action_values, act, q2)
